# baseline (speedup 1.0000x reference)
import functools

import jax
import jax.numpy as jnp
from jax import lax
from jax.experimental import pallas as pl
from jax.experimental.pallas import tpu as pltpu


def _p1_kernel(a_ref, x_ref, wt_ref, yt_ref, din_ref, doutp_ref, a8_ref):
    k = pl.program_id(1)
    a = a_ref[...]
    x = x_ref[...]

    yt_ref[...] = lax.dot_general(
        wt_ref[...], x, (((1,), (1,)), ((), ())),
        preferred_element_type=jnp.float32)

    ones_row = jnp.ones((1, a.shape[1]), jnp.float32)
    din_ref[...] = lax.dot_general(
        ones_row, a, (((1,), (1,)), ((), ())),
        preferred_element_type=jnp.float32)

    @pl.when(k == 0)
    def _():
        doutp_ref[...] = jnp.zeros_like(doutp_ref)

    ones_col = jnp.ones((1, a.shape[0]), jnp.float32)
    doutp_ref[0] += lax.dot_general(
        ones_col, a, (((1,), (0,)), ((), ())),
        preferred_element_type=jnp.float32)

    a8_ref[...] = a.astype(jnp.int8)


def _p2_kernel(a8_ref, ytk_ref, doutp_ref, ytm_ref, din_ref, b1_ref, b2_ref,
               score_ref, scol_ref, acc_ref):
    k = pl.program_id(1)

    @pl.when(k == 0)
    def _():
        acc_ref[...] = jnp.zeros_like(acc_ref)

    a = a8_ref[...].astype(jnp.float32)
    dout = doutp_ref[0] + doutp_ref[1]
    inv_out = lax.rsqrt(jnp.maximum(dout, 1.0))
    z = inv_out * ytk_ref[0:1, :]
    tc = min(1024, a.shape[1])
    acc = acc_ref[...]
    for t in range(a.shape[1] // tc):
        acc = acc + lax.dot_general(
            z[:, t * tc:(t + 1) * tc], a[:, t * tc:(t + 1) * tc],
            (((1,), (1,)), ((), ())),
            preferred_element_type=jnp.float32)
    acc_ref[...] = acc

    @pl.when(k == pl.num_programs(1) - 1)
    def _():
        inv_in = lax.rsqrt(jnp.maximum(din_ref[...], 1.0))
        s1 = inv_in * acc_ref[...] + b1_ref[0, 0]
        s2 = ytm_ref[1:2, :] + b2_ref[0, 0]
        s = jnp.maximum(s1, s2)
        score_ref[...] = s.reshape(score_ref.shape)
        scol_ref[...] = jnp.transpose(s)


def _subadj_kernel(a8_ref, permf_ref, permg_ref, x_ref, sc_ref,
                   o_ref, pool_ref, *, seg, kk, n_graphs):
    i = pl.program_id(0)
    k = pl.program_id(1)
    nsb = pl.num_programs(1)
    a = a8_ref[...]
    iota = lax.broadcasted_iota(jnp.int32, (seg, kk), 0)
    g_off = (i * nsb + k) * seg
    pg = permg_ref[...] - g_off
    mask = iota == pg
    ohr = mask.astype(jnp.float32)
    rsel = lax.dot_general(
        mask.astype(jnp.int8), a, (((0,), (0,)), ((), ())),
        preferred_element_type=jnp.int32)
    rsel = rsel.astype(jnp.bfloat16)
    cols = []
    for g in range(n_graphs):
        pgc = permf_ref[0:1, g * kk:(g + 1) * kk] - g * seg
        oh = (iota == pgc).astype(jnp.bfloat16)
        cols.append(lax.dot_general(
            rsel[:, g * seg:(g + 1) * seg], oh,
            (((1,), (0,)), ((), ())),
            preferred_element_type=jnp.float32))
    o_ref[...] = jnp.concatenate(cols, axis=1)
    px = lax.dot_general(
        ohr, x_ref[...], (((0,), (0,)), ((), ())),
        precision=lax.Precision.HIGHEST,
        preferred_element_type=jnp.float32)
    ps = lax.dot_general(
        ohr, sc_ref[...], (((0,), (0,)), ((), ())),
        precision=lax.Precision.HIGHEST,
        preferred_element_type=jnp.float32)
    pool_ref[...] = px * jnp.tanh(ps)


def kernel(adj, feature, w1, b1, w2, b2):
    n, d = feature.shape
    n_graphs = 8
    seg = n // n_graphs
    kk = -(-seg // 2)

    wt = jnp.concatenate([w1, w2], axis=1).T.astype(jnp.float32)

    nbk = 4
    rb = n // (2 * nbk)
    yt, din, doutp, a8 = pl.pallas_call(
        _p1_kernel,
        out_shape=(
            jax.ShapeDtypeStruct((2, n), jnp.float32),
            jax.ShapeDtypeStruct((1, n), jnp.float32),
            jax.ShapeDtypeStruct((2, 1, n), jnp.float32),
            jax.ShapeDtypeStruct((n, n), jnp.int8),
        ),
        grid=(2, nbk),
        in_specs=[
            pl.BlockSpec((rb, n), lambda i, k: (i * nbk + k, 0)),
            pl.BlockSpec((rb, d), lambda i, k: (i * nbk + k, 0)),
            pl.BlockSpec((2, d), lambda i, k: (0, 0)),
        ],
        out_specs=(
            pl.BlockSpec((2, rb), lambda i, k: (0, i * nbk + k)),
            pl.BlockSpec((1, rb), lambda i, k: (0, i * nbk + k)),
            pl.BlockSpec((1, 1, n), lambda i, k: (i, 0, 0)),
            pl.BlockSpec((rb, n), lambda i, k: (i * nbk + k, 0)),
        ),
        compiler_params=pltpu.CompilerParams(
            dimension_semantics=("parallel", "arbitrary")),
    )(adj, feature, wt)

    m = n // 2
    tk = n
    score_g, score_col = pl.pallas_call(
        _p2_kernel,
        out_shape=(jax.ShapeDtypeStruct((n_graphs, 1, seg), jnp.float32),
                   jax.ShapeDtypeStruct((n, 1), jnp.float32)),
        grid=(2, n // tk),
        in_specs=[
            pl.BlockSpec((m, tk), lambda i, k: (i, k)),
            pl.BlockSpec((2, tk), lambda i, k: (0, k)),
            pl.BlockSpec((2, 1, tk), lambda i, k: (0, 0, k)),
            pl.BlockSpec((2, m), lambda i, k: (0, i)),
            pl.BlockSpec((1, m), lambda i, k: (0, i)),
            pl.BlockSpec(memory_space=pltpu.MemorySpace.SMEM),
            pl.BlockSpec(memory_space=pltpu.MemorySpace.SMEM),
        ],
        out_specs=(pl.BlockSpec((n_graphs // 2, 1, seg), lambda i, k: (i, 0, 0)),
                   pl.BlockSpec((m, 1), lambda i, k: (i, 0))),
        scratch_shapes=[pltpu.VMEM((1, m), jnp.float32)],
        compiler_params=pltpu.CompilerParams(
            dimension_semantics=("parallel", "arbitrary")),
    )(a8, yt, doutp, yt, din, b1, b2)

    _, idx = lax.top_k(score_g.reshape(n_graphs, seg), kk)
    offs = (seg * jnp.arange(n_graphs, dtype=idx.dtype))[:, None]
    perm = (idx + offs).reshape(-1)

    big_k = perm.shape[0]
    nsb = n_graphs // 2
    perm2d = perm.reshape(1, big_k)
    sub_adj, pooled = pl.pallas_call(
        functools.partial(_subadj_kernel, seg=seg, kk=kk, n_graphs=n_graphs),
        out_shape=(jax.ShapeDtypeStruct((big_k, big_k), jnp.float32),
                   jax.ShapeDtypeStruct((big_k, d), jnp.float32)),
        grid=(2, nsb),
        in_specs=[
            pl.BlockSpec((seg, n), lambda i, k: (i * nsb + k, 0)),
            pl.BlockSpec((1, big_k), lambda i, k: (0, 0)),
            pl.BlockSpec((1, kk), lambda i, k: (0, i * nsb + k)),
            pl.BlockSpec((seg, d), lambda i, k: (i * nsb + k, 0)),
            pl.BlockSpec((seg, 1), lambda i, k: (i * nsb + k, 0)),
        ],
        out_specs=(
            pl.BlockSpec((kk, big_k), lambda i, k: (i * nsb + k, 0)),
            pl.BlockSpec((kk, d), lambda i, k: (i * nsb + k, 0)),
        ),
        compiler_params=pltpu.CompilerParams(
            dimension_semantics=("parallel", "arbitrary")),
    )(a8, perm2d, perm2d, feature, score_col)
    return sub_adj, pooled, perm, [kk] * n_graphs


# speedup vs baseline: 1.0082x; 1.0082x over previous
import functools

import jax
import jax.numpy as jnp
from jax import lax
from jax.experimental import pallas as pl
from jax.experimental.pallas import tpu as pltpu


def _p1_kernel(a_ref, x_ref, wt_ref, yt_ref, din_ref, doutp_ref, a8_ref):
    k = pl.program_id(1)
    a = a_ref[...]
    x = x_ref[...]

    yt_ref[...] = lax.dot_general(
        wt_ref[...], x, (((1,), (1,)), ((), ())),
        preferred_element_type=jnp.float32)

    ones_row = jnp.ones((1, a.shape[1]), jnp.float32)
    din_ref[...] = lax.dot_general(
        ones_row, a, (((1,), (1,)), ((), ())),
        preferred_element_type=jnp.float32)

    @pl.when(k == 0)
    def _():
        doutp_ref[...] = jnp.zeros_like(doutp_ref)

    ones_col = jnp.ones((1, a.shape[0]), jnp.float32)
    doutp_ref[0] += lax.dot_general(
        ones_col, a, (((1,), (0,)), ((), ())),
        preferred_element_type=jnp.float32)

    a8_ref[...] = a.astype(jnp.int8)


def _p2_kernel(a8_ref, ytk_ref, doutp_ref, ytm_ref, din_ref, b1_ref, b2_ref,
               score_ref, scol_ref, acc_ref):
    k = pl.program_id(1)

    @pl.when(k == 0)
    def _():
        acc_ref[...] = jnp.zeros_like(acc_ref)

    a = a8_ref[...].astype(jnp.float32)
    dout = doutp_ref[0] + doutp_ref[1]
    inv_out = lax.rsqrt(jnp.maximum(dout, 1.0))
    z = inv_out * ytk_ref[0:1, :]
    tc = min(1024, a.shape[1])
    acc = acc_ref[...]
    for t in range(a.shape[1] // tc):
        acc = acc + lax.dot_general(
            z[:, t * tc:(t + 1) * tc], a[:, t * tc:(t + 1) * tc],
            (((1,), (1,)), ((), ())),
            preferred_element_type=jnp.float32)
    acc_ref[...] = acc

    @pl.when(k == pl.num_programs(1) - 1)
    def _():
        inv_in = lax.rsqrt(jnp.maximum(din_ref[...], 1.0))
        s1 = inv_in * acc_ref[...] + b1_ref[0, 0]
        s2 = ytm_ref[1:2, :] + b2_ref[0, 0]
        s = jnp.maximum(s1, s2)
        score_ref[...] = s.reshape(score_ref.shape)
        scol_ref[...] = jnp.transpose(s)


def _subadj_kernel(a8_ref, permf_ref, permg_ref, x_ref, sc_ref,
                   o_ref, pool_ref, *, seg, kk, n_graphs):
    i = pl.program_id(0)
    k = pl.program_id(1)
    nsb = pl.num_programs(1)
    a = a8_ref[...]
    iota = lax.broadcasted_iota(jnp.int32, (seg, kk), 0)
    g_off = (i * nsb + k) * seg
    pg = permg_ref[...] - g_off
    mask = iota == pg
    ohr = mask.astype(jnp.float32)
    rsel = lax.dot_general(
        mask.astype(jnp.int8), a, (((0,), (0,)), ((), ())),
        preferred_element_type=jnp.int32)
    rsel = rsel.astype(jnp.bfloat16)
    cols = []
    for g in range(n_graphs):
        pgc = permf_ref[0:1, g * kk:(g + 1) * kk] - g * seg
        oh = (iota == pgc).astype(jnp.bfloat16)
        cols.append(lax.dot_general(
            rsel[:, g * seg:(g + 1) * seg], oh,
            (((1,), (0,)), ((), ())),
            preferred_element_type=jnp.float32))
    o_ref[...] = jnp.concatenate(cols, axis=1)
    px = lax.dot_general(
        ohr, x_ref[...], (((0,), (0,)), ((), ())),
        precision=lax.Precision.HIGHEST,
        preferred_element_type=jnp.float32)
    ps = lax.dot_general(
        ohr, sc_ref[...], (((0,), (0,)), ((), ())),
        precision=lax.Precision.HIGHEST,
        preferred_element_type=jnp.float32)
    pool_ref[...] = px * jnp.tanh(ps)


def kernel(adj, feature, w1, b1, w2, b2):
    n, d = feature.shape
    n_graphs = 8
    seg = n // n_graphs
    kk = -(-seg // 2)

    wt = jnp.concatenate([w1, w2], axis=1).T.astype(jnp.float32)

    nbk = 4
    rb = n // (2 * nbk)
    yt, din, doutp, a8 = pl.pallas_call(
        _p1_kernel,
        out_shape=(
            jax.ShapeDtypeStruct((2, n), jnp.float32),
            jax.ShapeDtypeStruct((1, n), jnp.float32),
            jax.ShapeDtypeStruct((2, 1, n), jnp.float32),
            jax.ShapeDtypeStruct((n, n), jnp.int8),
        ),
        grid=(2, nbk),
        in_specs=[
            pl.BlockSpec((rb, n), lambda i, k: (i * nbk + k, 0)),
            pl.BlockSpec((rb, d), lambda i, k: (i * nbk + k, 0)),
            pl.BlockSpec((2, d), lambda i, k: (0, 0)),
        ],
        out_specs=(
            pl.BlockSpec((2, rb), lambda i, k: (0, i * nbk + k)),
            pl.BlockSpec((1, rb), lambda i, k: (0, i * nbk + k)),
            pl.BlockSpec((1, 1, n), lambda i, k: (i, 0, 0)),
            pl.BlockSpec((rb, n), lambda i, k: (i * nbk + k, 0)),
        ),
        compiler_params=pltpu.CompilerParams(
            dimension_semantics=("parallel", "arbitrary")),
    )(adj, feature, wt)

    m = n // 2
    tk = min(2048, n)
    score_g, score_col = pl.pallas_call(
        _p2_kernel,
        out_shape=(jax.ShapeDtypeStruct((n_graphs, 1, seg), jnp.float32),
                   jax.ShapeDtypeStruct((n, 1), jnp.float32)),
        grid=(2, n // tk),
        in_specs=[
            pl.BlockSpec((m, tk), lambda i, k: (i, k)),
            pl.BlockSpec((2, tk), lambda i, k: (0, k)),
            pl.BlockSpec((2, 1, tk), lambda i, k: (0, 0, k)),
            pl.BlockSpec((2, m), lambda i, k: (0, i)),
            pl.BlockSpec((1, m), lambda i, k: (0, i)),
            pl.BlockSpec(memory_space=pltpu.MemorySpace.SMEM),
            pl.BlockSpec(memory_space=pltpu.MemorySpace.SMEM),
        ],
        out_specs=(pl.BlockSpec((n_graphs // 2, 1, seg), lambda i, k: (i, 0, 0)),
                   pl.BlockSpec((m, 1), lambda i, k: (i, 0))),
        scratch_shapes=[pltpu.VMEM((1, m), jnp.float32)],
        compiler_params=pltpu.CompilerParams(
            dimension_semantics=("parallel", "arbitrary")),
    )(a8, yt, doutp, yt, din, b1, b2)

    _, idx = lax.top_k(score_g.reshape(n_graphs, seg), kk)
    offs = (seg * jnp.arange(n_graphs, dtype=idx.dtype))[:, None]
    perm = (idx + offs).reshape(-1)

    big_k = perm.shape[0]
    nsb = n_graphs // 2
    perm2d = perm.reshape(1, big_k)
    sub_adj, pooled = pl.pallas_call(
        functools.partial(_subadj_kernel, seg=seg, kk=kk, n_graphs=n_graphs),
        out_shape=(jax.ShapeDtypeStruct((big_k, big_k), jnp.float32),
                   jax.ShapeDtypeStruct((big_k, d), jnp.float32)),
        grid=(2, nsb),
        in_specs=[
            pl.BlockSpec((seg, n), lambda i, k: (i * nsb + k, 0)),
            pl.BlockSpec((1, big_k), lambda i, k: (0, 0)),
            pl.BlockSpec((1, kk), lambda i, k: (0, i * nsb + k)),
            pl.BlockSpec((seg, d), lambda i, k: (i * nsb + k, 0)),
            pl.BlockSpec((seg, 1), lambda i, k: (i * nsb + k, 0)),
        ],
        out_specs=(
            pl.BlockSpec((kk, big_k), lambda i, k: (i * nsb + k, 0)),
            pl.BlockSpec((kk, d), lambda i, k: (i * nsb + k, 0)),
        ),
        compiler_params=pltpu.CompilerParams(
            dimension_semantics=("parallel", "arbitrary")),
    )(a8, perm2d, perm2d, feature, score_col)
    return sub_adj, pooled, perm, [kk] * n_graphs
